# baseline (speedup 1.0000x reference)
import functools

import jax
import jax.numpy as jnp
from jax import lax
from jax.experimental import pallas as pl
from jax.experimental.pallas import tpu as pltpu
from jax.experimental.pallas import tpu_sc as plsc

_NC = 2
_NS = 16
_NW = _NC * _NS
_L = 16
_G = 16
_UBC = 64
_IBC = 72


def _mf_kernel(B, F, u_emb3, i_emb3, u_bias3, i_bias3, u_idx, i_idx):
    b_per_w = B // _NW
    ng = b_per_w // _G
    group_bytes = 2 * _G * (F + 1) * 4
    drain_words = group_bytes // 4
    mesh = plsc.VectorSubcoreMesh(core_axis_name="c", subcore_axis_name="s")
    cp = pltpu.CompilerParams(needs_layout_passes=False)

    @functools.partial(
        pl.kernel,
        mesh=mesh,
        compiler_params=cp,
        out_type=jax.ShapeDtypeStruct((B,), jnp.float32),
        scratch_types=[
            pltpu.VMEM((b_per_w,), jnp.int32),
            pltpu.VMEM((b_per_w,), jnp.int32),
            pltpu.VMEM((2, _G, 128), jnp.float32),
            pltpu.VMEM((2, _G, 128), jnp.float32),
            pltpu.VMEM((drain_words,), jnp.int32),
            pltpu.VMEM((b_per_w,), jnp.float32),
            pltpu.SemaphoreType.DMA,
            pltpu.SemaphoreType.DMA,
        ],
    )
    def k(u3, i3, ub3, ib3, u_idx_hbm, i_idx_hbm,
          out_hbm, uidx_v, iidx_v, u_ring, i_ring, drain_v, out_v,
          sem0, sem1):
        wid = lax.axis_index("s") * _NC + lax.axis_index("c")
        base = wid * b_per_w

        pltpu.sync_copy(u_idx_hbm.at[pl.ds(base, b_per_w)], uidx_v)
        pltpu.sync_copy(i_idx_hbm.at[pl.ds(base, b_per_w)], iidx_v)

        lane = lax.iota(jnp.int32, _L)
        bias_mask = (lane == 0) | (lane == 8)

        def enqueue(g, buf, sem):
            rb = g * _G
            uidx16 = uidx_v[pl.ds(rb, _G)]
            iidx16 = iidx_v[pl.ds(rb, _G)]
            ug16 = lax.shift_right_logical(uidx16, 3)
            us16 = lax.bitwise_and(uidx16, 7)
            ig16 = lax.shift_right_logical(iidx16, 3)
            is16 = lax.bitwise_and(iidx16, 7)
            for j in range(_G):
                ug, us = ug16[j], us16[j]
                ig, is_ = ig16[j], is16[j]
                pltpu.async_copy(u3.at[ug, us], u_ring.at[buf, j, pl.ds(0, F)], sem)
                pltpu.async_copy(i3.at[ig, is_], i_ring.at[buf, j, pl.ds(0, F)], sem)
                pltpu.async_copy(ub3.at[ug, us], u_ring.at[buf, j, pl.ds(_UBC, 1)], sem)
                pltpu.async_copy(ib3.at[ig, is_], u_ring.at[buf, j, pl.ds(_IBC, 1)], sem)

        def drain(sem):
            pltpu.make_async_copy(
                u_idx_hbm.at[pl.ds(0, drain_words)], drain_v, sem).wait()

        def compute(g, buf):
            rb = g * _G
            out16 = jnp.zeros((_L,), jnp.float32)
            for j in range(_G):
                acc = (u_ring[buf, j, pl.ds(0, _L)]
                       * i_ring[buf, j, pl.ds(0, _L)])
                for fb in range(1, F // _L):
                    acc = acc + (u_ring[buf, j, pl.ds(fb * _L, _L)]
                                 * i_ring[buf, j, pl.ds(fb * _L, _L)])
                bias16 = u_ring[buf, j, pl.ds(_UBC, _L)]
                acc = acc + jnp.where(bias_mask, bias16, 0.0)
                out16 = out16 + jnp.where(lane == j, jnp.sum(acc), 0.0)
            out_v[pl.ds(rb, _L)] = out16

        enqueue(0, 0, sem0)

        @pl.loop(0, ng // 2)
        def _(kk):
            g0 = kk * 2
            enqueue(g0 + 1, 1, sem1)
            drain(sem0)
            compute(g0, 0)

            @pl.when(g0 + 2 < ng)
            def _():
                enqueue(g0 + 2, 0, sem0)

            drain(sem1)
            compute(g0 + 1, 1)

        pltpu.sync_copy(out_v, out_hbm.at[pl.ds(base, b_per_w)])

    return k(u_emb3, i_emb3, u_bias3, i_bias3, u_idx, i_idx)


@jax.jit
def kernel(u_emb, i_emb, u_bias, i_bias, u_idx, i_idx):
    B = u_idx.shape[0]
    F = u_emb.shape[1]
    u3 = u_emb.reshape(-1, 8, F)
    i3 = i_emb.reshape(-1, 8, F)
    ub3 = u_bias.reshape(-1, 8, 1)
    ib3 = i_bias.reshape(-1, 8, 1)
    return _mf_kernel(
        B, F, u3, i3, ub3, ib3,
        u_idx.astype(jnp.int32), i_idx.astype(jnp.int32),
    )


# speedup vs baseline: 3.0066x; 3.0066x over previous
import functools

import jax
import jax.numpy as jnp
from jax import lax
from jax.experimental import pallas as pl
from jax.experimental.pallas import tpu as pltpu
from jax.experimental.pallas import tpu_sc as plsc

_NC = 2
_NS = 16
_NW = _NC * _NS
_L = 16
_G = 16


def _mf_kernel(B, F, u_emb3, i_emb3, u_bias1, i_bias1, u_idx, i_idx):
    b_per_w = B // _NW
    ng = b_per_w // _G
    nrow = b_per_w // 128
    group_bytes = 2 * _G * F * 4
    drain_words = group_bytes // 4
    mesh = plsc.VectorSubcoreMesh(core_axis_name="c", subcore_axis_name="s")
    cp = pltpu.CompilerParams(needs_layout_passes=False)

    @functools.partial(
        pl.kernel,
        mesh=mesh,
        compiler_params=cp,
        out_type=jax.ShapeDtypeStruct((B,), jnp.float32),
        scratch_types=[
            pltpu.VMEM((b_per_w // 128, 128), jnp.int32),
            pltpu.VMEM((b_per_w // 128, 128), jnp.int32),
            pltpu.VMEM((2, _G, 128), jnp.float32),
            pltpu.VMEM((2, _G, 128), jnp.float32),
            pltpu.VMEM((b_per_w,), jnp.float32),
            pltpu.VMEM((b_per_w,), jnp.float32),
            pltpu.VMEM((drain_words,), jnp.int32),
            pltpu.VMEM((b_per_w,), jnp.float32),
            pltpu.SemaphoreType.DMA,
            pltpu.SemaphoreType.DMA,
            pltpu.SemaphoreType.DMA,
        ],
    )
    def k(u3, i3, ub_hbm, ib_hbm, u_idx_hbm, i_idx_hbm,
          out_hbm, uidx_v, iidx_v, u_ring, i_ring, ub_v, ib_v, drain_v,
          out_v, sem0, sem1, bsem):
        wid = lax.axis_index("s") * _NC + lax.axis_index("c")
        base = wid * b_per_w

        for kk in range(nrow):
            pltpu.sync_copy(u_idx_hbm.at[pl.ds(base + kk * 128, 128)],
                            uidx_v.at[kk])
            pltpu.sync_copy(i_idx_hbm.at[pl.ds(base + kk * 128, 128)],
                            iidx_v.at[kk])

        for kk in range(nrow):
            pltpu.async_copy(ub_hbm.at[uidx_v.at[kk]],
                             ub_v.at[pl.ds(kk * 128, 128)], bsem)
            pltpu.async_copy(ib_hbm.at[iidx_v.at[kk]],
                             ib_v.at[pl.ds(kk * 128, 128)], bsem)

        def enqueue(g, buf, sem):
            rb = g * _G
            uidx16 = uidx_v[rb // 128, pl.ds(rb % 128, _G)]
            iidx16 = iidx_v[rb // 128, pl.ds(rb % 128, _G)]
            ug16 = lax.shift_right_logical(uidx16, 3)
            us16 = lax.bitwise_and(uidx16, 7)
            ig16 = lax.shift_right_logical(iidx16, 3)
            is16 = lax.bitwise_and(iidx16, 7)
            for j in range(_G):
                ug, us = ug16[j], us16[j]
                ig, is_ = ig16[j], is16[j]
                pltpu.async_copy(u3.at[ug, us], u_ring.at[buf, j, pl.ds(0, F)], sem)
                pltpu.async_copy(i3.at[ig, is_], i_ring.at[buf, j, pl.ds(0, F)], sem)

        def drain(sem):
            pltpu.make_async_copy(
                u_idx_hbm.at[pl.ds(0, drain_words)], drain_v, sem).wait()

        lane = lax.iota(jnp.int32, _L)

        def compute(g, buf):
            rb = g * _G
            out16 = ub_v[pl.ds(rb, _G)] + ib_v[pl.ds(rb, _G)]
            for j in range(_G):
                acc = (u_ring[buf, j, pl.ds(0, _L)]
                       * i_ring[buf, j, pl.ds(0, _L)])
                for fb in range(1, F // _L):
                    acc = acc + (u_ring[buf, j, pl.ds(fb * _L, _L)]
                                 * i_ring[buf, j, pl.ds(fb * _L, _L)])
                out16 = out16 + jnp.where(lane == j, jnp.sum(acc), 0.0)
            out_v[pl.ds(rb, _G)] = out16

        enqueue(0, 0, sem0)
        for kk in range(2 * nrow):
            pltpu.make_async_copy(
                u_idx_hbm.at[pl.ds(0, 128)],
                drain_v.at[pl.ds(0, 128)], bsem).wait()

        @pl.loop(0, ng // 2)
        def _(kk):
            g0 = kk * 2
            enqueue(g0 + 1, 1, sem1)
            drain(sem0)
            compute(g0, 0)

            @pl.when(g0 + 2 < ng)
            def _():
                enqueue(g0 + 2, 0, sem0)

            drain(sem1)
            compute(g0 + 1, 1)

        pltpu.sync_copy(out_v, out_hbm.at[pl.ds(base, b_per_w)])

    return k(u_emb3, i_emb3, u_bias1, i_bias1, u_idx, i_idx)


@jax.jit
def kernel(u_emb, i_emb, u_bias, i_bias, u_idx, i_idx):
    B = u_idx.shape[0]
    F = u_emb.shape[1]
    u3 = u_emb.reshape(-1, 8, F)
    i3 = i_emb.reshape(-1, 8, F)
    ub1 = u_bias.reshape(-1)
    ib1 = i_bias.reshape(-1)
    return _mf_kernel(
        B, F, u3, i3, ub1, ib1,
        u_idx.astype(jnp.int32), i_idx.astype(jnp.int32),
    )
